# initial kernel scaffold (unmeasured)
import functools

import jax
import jax.numpy as jnp
from jax import lax
from jax.experimental import pallas as pl
from jax.experimental.pallas import tpu as pltpu

N_DEV = 4
N_EXP_LOCAL = 8


def _neighbor_barrier(left, right):
    barrier = pltpu.get_barrier_semaphore()
    for nbr in (left, right):
        pl.semaphore_signal(
            barrier, inc=1, device_id=(nbr,),
            device_id_type=pl.DeviceIdType.MESH,
        )
    pl.semaphore_wait(barrier, 2)


def _exit_barrier(left, right):
    @functools.partial(pl.run_scoped, sem=pltpu.SemaphoreType.REGULAR)
    def _(sem):
        for nbr in (left, right):
            pl.semaphore_signal(
                sem, inc=1, device_id=(nbr,),
                device_id_type=pl.DeviceIdType.MESH,
            )
        pl.semaphore_wait(sem, 2)


def _ag_kernel(xb, meta):
    m, dmodel = xb.shape

    def body(x_ref, meta_ref, xall_ref, metaall_ref,
             comm_x, comm_m, sx, rx, sm, rm):
        d = lax.axis_index("i")
        left = lax.rem(d + N_DEV - 1, N_DEV)
        right = lax.rem(d + 1, N_DEV)

        _neighbor_barrier(left, right)

        comm_x[0, :, :] = x_ref[:, :]
        comm_m[0, :, :] = meta_ref[:, :]
        xall_ref[pl.ds(d * m, m), :] = x_ref[:, :]
        metaall_ref[:, pl.ds(d * m, m)] = meta_ref[:, :]

        for h in range(N_DEV - 1):
            cp_x = pltpu.make_async_remote_copy(
                src_ref=comm_x.at[h], dst_ref=comm_x.at[h + 1],
                send_sem=sx.at[h], recv_sem=rx.at[h],
                device_id=(right,), device_id_type=pl.DeviceIdType.MESH,
            )
            cp_m = pltpu.make_async_remote_copy(
                src_ref=comm_m.at[h], dst_ref=comm_m.at[h + 1],
                send_sem=sm.at[h], recv_sem=rm.at[h],
                device_id=(right,), device_id_type=pl.DeviceIdType.MESH,
            )
            cp_x.start()
            cp_m.start()
            cp_x.wait()
            cp_m.wait()
            origin = lax.rem(d + N_DEV - h - 1, N_DEV)
            xall_ref[pl.ds(origin * m, m), :] = comm_x[h + 1]
            metaall_ref[:, pl.ds(origin * m, m)] = comm_m[h + 1]

        _exit_barrier(left, right)

    return pl.pallas_call(
        body,
        out_shape=(
            jax.ShapeDtypeStruct((N_DEV * m, dmodel), jnp.bfloat16),
            jax.ShapeDtypeStruct((2, N_DEV * m), jnp.float32),
        ),
        in_specs=[
            pl.BlockSpec(memory_space=pltpu.VMEM),
            pl.BlockSpec(memory_space=pltpu.VMEM),
        ],
        out_specs=(
            pl.BlockSpec(memory_space=pltpu.VMEM),
            pl.BlockSpec(memory_space=pltpu.VMEM),
        ),
        scratch_shapes=[
            pltpu.VMEM((N_DEV, m, dmodel), jnp.bfloat16),
            pltpu.VMEM((N_DEV, 2, m), jnp.float32),
            pltpu.SemaphoreType.DMA((N_DEV - 1,)),
            pltpu.SemaphoreType.DMA((N_DEV - 1,)),
            pltpu.SemaphoreType.DMA((N_DEV - 1,)),
            pltpu.SemaphoreType.DMA((N_DEV - 1,)),
        ],
        compiler_params=pltpu.CompilerParams(collective_id=0),
    )(xb, meta)


def _compute_kernel(x_all, meta_all, ew, sw):
    n_tok, dmodel = x_all.shape
    m = n_tok // N_DEV
    hdim = ew.shape[-1]

    def body(x_ref, meta_ref, ew_ref, sw_ref, out_ref):
        c = pl.program_id(0)
        j = pl.program_id(1)
        d = lax.axis_index("i")
        idx = meta_ref[0, :]
        ps = meta_ref[1, :]
        gid = (d * N_EXP_LOCAL + j).astype(jnp.float32)
        scale = jnp.where(idx == gid, ps, 0.0).astype(jnp.bfloat16)
        xb = x_ref[:, :]
        contrib = lax.dot_general(
            xb * scale[:, None], ew_ref[0],
            (((1,), (0,)), ((), ())),
            preferred_element_type=jnp.float32,
        )

        @pl.when(j == 0)
        def _():
            out_ref[:, :] = contrib.astype(jnp.bfloat16)

        @pl.when(j > 0)
        def _():
            out_ref[:, :] = (
                out_ref[:, :].astype(jnp.float32) + contrib
            ).astype(jnp.bfloat16)

        @pl.when(jnp.logical_and(j == N_EXP_LOCAL - 1, c == d))
        def _():
            shared = lax.dot_general(
                xb, sw_ref[:, :], (((1,), (0,)), ((), ())),
                preferred_element_type=jnp.float32,
            )
            out_ref[:, :] = (
                out_ref[:, :].astype(jnp.float32) + shared
            ).astype(jnp.bfloat16)

    return pl.pallas_call(
        body,
        grid=(N_DEV, N_EXP_LOCAL),
        out_shape=jax.ShapeDtypeStruct((n_tok, dmodel), jnp.bfloat16),
        in_specs=[
            pl.BlockSpec((m, dmodel), lambda c, j: (c, 0)),
            pl.BlockSpec((2, m), lambda c, j: (0, c)),
            pl.BlockSpec((1, dmodel, hdim), lambda c, j: (j, 0, 0)),
            pl.BlockSpec((dmodel, hdim), lambda c, j: (0, 0)),
        ],
        out_specs=pl.BlockSpec((m, dmodel), lambda c, j: (c, 0)),
        compiler_params=pltpu.CompilerParams(
            dimension_semantics=("arbitrary", "arbitrary"),
        ),
    )(x_all, meta_all, ew, sw)


def _rs_kernel(partial):
    n_tok, dmodel = partial.shape
    m = n_tok // N_DEV

    def body(p_ref, out_ref, comm, sendbuf, ss, rs):
        d = lax.axis_index("i")
        left = lax.rem(d + N_DEV - 1, N_DEV)
        right = lax.rem(d + 1, N_DEV)

        _neighbor_barrier(left, right)

        for s in range(N_DEV - 1):
            src = lax.rem(d + N_DEV - 1 - s, N_DEV)
            val = p_ref[pl.ds(src * m, m), :]
            if s > 0:
                val = val + comm[s - 1]
            sendbuf[:, :] = val
            cp = pltpu.make_async_remote_copy(
                src_ref=sendbuf, dst_ref=comm.at[s],
                send_sem=ss.at[s], recv_sem=rs.at[s],
                device_id=(right,), device_id_type=pl.DeviceIdType.MESH,
            )
            cp.start()
            cp.wait()

        mine = p_ref[pl.ds(d * m, m), :].astype(jnp.float32)
        out_ref[:, :] = mine + comm[N_DEV - 2].astype(jnp.float32)

        _exit_barrier(left, right)

    return pl.pallas_call(
        body,
        out_shape=jax.ShapeDtypeStruct((m, dmodel), jnp.float32),
        in_specs=[pl.BlockSpec(memory_space=pltpu.VMEM)],
        out_specs=pl.BlockSpec(memory_space=pltpu.VMEM),
        scratch_shapes=[
            pltpu.VMEM((N_DEV - 1, m, dmodel), jnp.bfloat16),
            pltpu.VMEM((m, dmodel), jnp.bfloat16),
            pltpu.SemaphoreType.DMA((N_DEV - 1,)),
            pltpu.SemaphoreType.DMA((N_DEV - 1,)),
        ],
        compiler_params=pltpu.CompilerParams(collective_id=1),
    )(partial)


def kernel(x, router_W, route_idx, expert_W, shared_W):
    xb = x.astype(jnp.bfloat16)
    ew = expert_W.astype(jnp.bfloat16)
    sw = shared_W.astype(jnp.bfloat16)

    scores = x @ router_W
    scores = scores - scores.max(axis=-1, keepdims=True)
    probs = jnp.exp(scores)
    probs = probs / probs.sum(axis=-1, keepdims=True)
    p_sel = jnp.take_along_axis(probs, route_idx, axis=1)[:, 0]
    meta = jnp.stack([route_idx[:, 0].astype(jnp.float32), p_sel])

    x_all, meta_all = _ag_kernel(xb, meta)
    partial = _compute_kernel(x_all, meta_all, ew, sw)
    return _rs_kernel(partial)


# baseline (device time: 526589 ns/iter reference)
import functools

import jax
import jax.numpy as jnp
from jax import lax
from jax.experimental import pallas as pl
from jax.experimental.pallas import tpu as pltpu

N_DEV = 4
N_EXP_LOCAL = 8


def _neighbor_barrier(left, right):
    barrier = pltpu.get_barrier_semaphore()
    for nbr in (left, right):
        pl.semaphore_signal(
            barrier, inc=1, device_id=(nbr,),
            device_id_type=pl.DeviceIdType.MESH,
        )
    pl.semaphore_wait(barrier, 2)


def _exit_barrier(left, right):
    @functools.partial(pl.run_scoped, sem=pltpu.SemaphoreType.REGULAR)
    def _(sem):
        for nbr in (left, right):
            pl.semaphore_signal(
                sem, inc=1, device_id=(nbr,),
                device_id_type=pl.DeviceIdType.MESH,
            )
        pl.semaphore_wait(sem, 2)


def _ag_kernel(xb, meta):
    m, dmodel = xb.shape

    def body(x_ref, meta_ref, xall_ref, metaall_ref,
             comm_x, comm_m, sx, rx, sm, rm):
        d = lax.axis_index("i")
        left = lax.rem(d + N_DEV - 1, N_DEV)
        right = lax.rem(d + 1, N_DEV)

        _neighbor_barrier(left, right)

        comm_x[0, :, :] = x_ref[:, :]
        comm_m[0, :, :] = meta_ref[:, :]
        xall_ref[pl.ds(d * m, m), :] = x_ref[:, :]
        metaall_ref[:, pl.ds(d * m, m)] = meta_ref[:, :]

        for h in range(N_DEV - 1):
            cp_x = pltpu.make_async_remote_copy(
                src_ref=comm_x.at[h], dst_ref=comm_x.at[h + 1],
                send_sem=sx.at[h], recv_sem=rx.at[h],
                device_id=(right,), device_id_type=pl.DeviceIdType.MESH,
            )
            cp_m = pltpu.make_async_remote_copy(
                src_ref=comm_m.at[h], dst_ref=comm_m.at[h + 1],
                send_sem=sm.at[h], recv_sem=rm.at[h],
                device_id=(right,), device_id_type=pl.DeviceIdType.MESH,
            )
            cp_x.start()
            cp_m.start()
            cp_x.wait()
            cp_m.wait()
            origin = lax.rem(d + N_DEV - h - 1, N_DEV)
            xall_ref[pl.ds(origin * m, m), :] = comm_x[h + 1]
            metaall_ref[:, pl.ds(origin * m, m)] = comm_m[h + 1]

        _exit_barrier(left, right)

    return pl.pallas_call(
        body,
        out_shape=(
            jax.ShapeDtypeStruct((N_DEV * m, dmodel), jnp.bfloat16),
            jax.ShapeDtypeStruct((2, N_DEV * m), jnp.float32),
        ),
        in_specs=[
            pl.BlockSpec(memory_space=pltpu.VMEM),
            pl.BlockSpec(memory_space=pltpu.VMEM),
        ],
        out_specs=(
            pl.BlockSpec(memory_space=pltpu.VMEM),
            pl.BlockSpec(memory_space=pltpu.VMEM),
        ),
        scratch_shapes=[
            pltpu.VMEM((N_DEV, m, dmodel), jnp.bfloat16),
            pltpu.VMEM((N_DEV, 2, m), jnp.float32),
            pltpu.SemaphoreType.DMA((N_DEV - 1,)),
            pltpu.SemaphoreType.DMA((N_DEV - 1,)),
            pltpu.SemaphoreType.DMA((N_DEV - 1,)),
            pltpu.SemaphoreType.DMA((N_DEV - 1,)),
        ],
        compiler_params=pltpu.CompilerParams(
            collective_id=0, vmem_limit_bytes=60 * 1024 * 1024,
        ),
    )(xb, meta)


def _compute_kernel(x_all, meta_all, ew, sw):
    n_tok, dmodel = x_all.shape
    m = n_tok // N_DEV
    hdim = ew.shape[-1]

    def body(x_ref, meta_ref, ew_ref, sw_ref, out_ref):
        c = pl.program_id(0)
        j = pl.program_id(1)
        d = lax.axis_index("i")
        idx = meta_ref[0, :]
        ps = meta_ref[1, :]
        gid = (d * N_EXP_LOCAL + j).astype(jnp.float32)
        scale = jnp.where(idx == gid, ps, 0.0).astype(jnp.bfloat16)
        xb = x_ref[:, :]
        contrib = lax.dot_general(
            xb * scale[:, None], ew_ref[0],
            (((1,), (0,)), ((), ())),
            preferred_element_type=jnp.float32,
        )

        @pl.when(j == 0)
        def _():
            out_ref[:, :] = contrib.astype(jnp.bfloat16)

        @pl.when(j > 0)
        def _():
            out_ref[:, :] = (
                out_ref[:, :].astype(jnp.float32) + contrib
            ).astype(jnp.bfloat16)

        @pl.when(jnp.logical_and(j == N_EXP_LOCAL - 1, c == d))
        def _():
            shared = lax.dot_general(
                xb, sw_ref[:, :], (((1,), (0,)), ((), ())),
                preferred_element_type=jnp.float32,
            )
            out_ref[:, :] = (
                out_ref[:, :].astype(jnp.float32) + shared
            ).astype(jnp.bfloat16)

    return pl.pallas_call(
        body,
        grid=(N_DEV, N_EXP_LOCAL),
        out_shape=jax.ShapeDtypeStruct((n_tok, dmodel), jnp.bfloat16),
        in_specs=[
            pl.BlockSpec((m, dmodel), lambda c, j: (c, 0)),
            pl.BlockSpec((2, m), lambda c, j: (0, c)),
            pl.BlockSpec((1, dmodel, hdim), lambda c, j: (j, 0, 0)),
            pl.BlockSpec((dmodel, hdim), lambda c, j: (0, 0)),
        ],
        out_specs=pl.BlockSpec((m, dmodel), lambda c, j: (c, 0)),
        compiler_params=pltpu.CompilerParams(
            dimension_semantics=("arbitrary", "arbitrary"),
            vmem_limit_bytes=60 * 1024 * 1024,
        ),
    )(x_all, meta_all, ew, sw)


def _rs_kernel(partial):
    n_tok, dmodel = partial.shape
    m = n_tok // N_DEV

    def body(p_ref, out_ref, comm, sendbuf, ss, rs):
        d = lax.axis_index("i")
        left = lax.rem(d + N_DEV - 1, N_DEV)
        right = lax.rem(d + 1, N_DEV)

        _neighbor_barrier(left, right)

        for s in range(N_DEV - 1):
            src = lax.rem(d + N_DEV - 1 - s, N_DEV)
            val = p_ref[pl.ds(src * m, m), :]
            if s > 0:
                val = val + comm[s - 1]
            sendbuf[:, :] = val
            cp = pltpu.make_async_remote_copy(
                src_ref=sendbuf, dst_ref=comm.at[s],
                send_sem=ss.at[s], recv_sem=rs.at[s],
                device_id=(right,), device_id_type=pl.DeviceIdType.MESH,
            )
            cp.start()
            cp.wait()

        mine = p_ref[pl.ds(d * m, m), :].astype(jnp.float32)
        out_ref[:, :] = mine + comm[N_DEV - 2].astype(jnp.float32)

        _exit_barrier(left, right)

    return pl.pallas_call(
        body,
        out_shape=jax.ShapeDtypeStruct((m, dmodel), jnp.float32),
        in_specs=[pl.BlockSpec(memory_space=pltpu.VMEM)],
        out_specs=pl.BlockSpec(memory_space=pltpu.VMEM),
        scratch_shapes=[
            pltpu.VMEM((N_DEV - 1, m, dmodel), jnp.bfloat16),
            pltpu.VMEM((m, dmodel), jnp.bfloat16),
            pltpu.SemaphoreType.DMA((N_DEV - 1,)),
            pltpu.SemaphoreType.DMA((N_DEV - 1,)),
        ],
        compiler_params=pltpu.CompilerParams(
            collective_id=1, vmem_limit_bytes=60 * 1024 * 1024,
        ),
    )(partial)


def kernel(x, router_W, route_idx, expert_W, shared_W):
    xb = x.astype(jnp.bfloat16)
    ew = expert_W.astype(jnp.bfloat16)
    sw = shared_W.astype(jnp.bfloat16)

    scores = x @ router_W
    scores = scores - scores.max(axis=-1, keepdims=True)
    probs = jnp.exp(scores)
    probs = probs / probs.sum(axis=-1, keepdims=True)
    p_sel = jnp.take_along_axis(probs, route_idx, axis=1)[:, 0]
    meta = jnp.stack([route_idx[:, 0].astype(jnp.float32), p_sel])

    x_all, meta_all = _ag_kernel(xb, meta)
    partial = _compute_kernel(x_all, meta_all, ew, sw)
    return _rs_kernel(partial)


# device time: 406560 ns/iter; 1.2952x vs baseline; 1.2952x over previous
import functools

import jax
import jax.numpy as jnp
from jax import lax
from jax.experimental import pallas as pl
from jax.experimental.pallas import tpu as pltpu

N_DEV = 4
N_EXP_LOCAL = 8
ROW_BLK = 1024


def _neighbor_barrier(left, right):
    barrier = pltpu.get_barrier_semaphore()
    for nbr in (left, right):
        pl.semaphore_signal(
            barrier, inc=1, device_id=(nbr,),
            device_id_type=pl.DeviceIdType.MESH,
        )
    pl.semaphore_wait(barrier, 2)


def _exit_barrier(left, right):
    @functools.partial(pl.run_scoped, sem=pltpu.SemaphoreType.REGULAR)
    def _(sem):
        for nbr in (left, right):
            pl.semaphore_signal(
                sem, inc=1, device_id=(nbr,),
                device_id_type=pl.DeviceIdType.MESH,
            )
        pl.semaphore_wait(sem, 2)


def _agc_kernel(xb, meta, ew, sw):
    m, dmodel = xb.shape
    hdim = ew.shape[-1]

    def body(x_ref, meta_ref, ew_ref, sw_ref, p_ref,
             comm_x, comm_m, sx, rx, sm, rm):
        d = lax.axis_index("i")
        left = lax.rem(d + N_DEV - 1, N_DEV)
        right = lax.rem(d + 1, N_DEV)

        _neighbor_barrier(left, right)

        def make_cp_x(h):
            return pltpu.make_async_remote_copy(
                src_ref=(x_ref if h == 0 else comm_x.at[h - 1]),
                dst_ref=comm_x.at[h],
                send_sem=sx.at[h], recv_sem=rx.at[h],
                device_id=(right,), device_id_type=pl.DeviceIdType.MESH,
            )

        def make_cp_m(h):
            return pltpu.make_async_remote_copy(
                src_ref=(meta_ref if h == 0 else comm_m.at[h - 1]),
                dst_ref=comm_m.at[h],
                send_sem=sm.at[h], recv_sem=rm.at[h],
                device_id=(right,), device_id_type=pl.DeviceIdType.MESH,
            )

        def compute_block(xblk, idx_row, ps_row, row_start, with_shared):
            def exp_step(j, acc):
                gid = (d * N_EXP_LOCAL + j).astype(jnp.float32)
                sc = jnp.where(idx_row == gid, ps_row, 0.0).astype(
                    jnp.bfloat16)
                wj = ew_ref[pl.ds(j, 1), :, :].reshape(dmodel, hdim)
                r = lax.dot_general(
                    xblk * sc[:, None], wj,
                    (((1,), (0,)), ((), ())),
                    preferred_element_type=jnp.float32,
                )
                return acc + r

            acc = lax.fori_loop(
                0, N_EXP_LOCAL, exp_step,
                jnp.zeros((ROW_BLK, hdim), jnp.float32),
            )
            if with_shared:
                acc = acc + lax.dot_general(
                    xblk, sw_ref[:, :], (((1,), (0,)), ((), ())),
                    preferred_element_type=jnp.float32,
                )
            p_ref[pl.ds(row_start, ROW_BLK), :] = acc.astype(jnp.bfloat16)

        def compute_chunk(x_get, idx_get, ps_get, origin, with_shared):
            for half in range(m // ROW_BLK):
                lo, hi = half * ROW_BLK, (half + 1) * ROW_BLK
                compute_block(
                    x_get(lo, hi), idx_get(lo, hi), ps_get(lo, hi),
                    origin * m + half * ROW_BLK, with_shared,
                )

        make_cp_x(0).start()
        make_cp_m(0).start()
        compute_chunk(
            lambda lo, hi: x_ref[lo:hi, :],
            lambda lo, hi: meta_ref[0, lo:hi],
            lambda lo, hi: meta_ref[1, lo:hi],
            d, True,
        )

        for h in range(N_DEV - 1):
            cpx, cpm = make_cp_x(h), make_cp_m(h)
            cpx.wait()
            cpm.wait()
            if h < N_DEV - 2:
                make_cp_x(h + 1).start()
                make_cp_m(h + 1).start()
            origin = lax.rem(d + N_DEV - h - 1, N_DEV)
            compute_chunk(
                lambda lo, hi, h=h: comm_x[h, lo:hi, :],
                lambda lo, hi, h=h: comm_m[h, 0, lo:hi],
                lambda lo, hi, h=h: comm_m[h, 1, lo:hi],
                origin, False,
            )

        _exit_barrier(left, right)

    return pl.pallas_call(
        body,
        out_shape=jax.ShapeDtypeStruct((N_DEV * m, dmodel), jnp.bfloat16),
        in_specs=[pl.BlockSpec(memory_space=pltpu.VMEM)] * 4,
        out_specs=pl.BlockSpec(memory_space=pltpu.VMEM),
        scratch_shapes=[
            pltpu.VMEM((N_DEV - 1, m, dmodel), jnp.bfloat16),
            pltpu.VMEM((N_DEV - 1, 2, m), jnp.float32),
            pltpu.SemaphoreType.DMA((N_DEV - 1,)),
            pltpu.SemaphoreType.DMA((N_DEV - 1,)),
            pltpu.SemaphoreType.DMA((N_DEV - 1,)),
            pltpu.SemaphoreType.DMA((N_DEV - 1,)),
        ],
        compiler_params=pltpu.CompilerParams(
            collective_id=0, vmem_limit_bytes=64 * 1024 * 1024,
        ),
    )(xb, meta, ew, sw)


def _rs_kernel(partial):
    n_tok, dmodel = partial.shape
    m = n_tok // N_DEV

    def body(p_ref, out_ref, comm, sendbuf, ss, rs):
        d = lax.axis_index("i")
        left = lax.rem(d + N_DEV - 1, N_DEV)
        right = lax.rem(d + 1, N_DEV)

        _neighbor_barrier(left, right)

        for s in range(N_DEV - 1):
            src = lax.rem(d + N_DEV - 1 - s, N_DEV)
            val = p_ref[pl.ds(src * m, m), :]
            if s > 0:
                val = val + comm[s - 1]
            sendbuf[:, :] = val
            cp = pltpu.make_async_remote_copy(
                src_ref=sendbuf, dst_ref=comm.at[s],
                send_sem=ss.at[s], recv_sem=rs.at[s],
                device_id=(right,), device_id_type=pl.DeviceIdType.MESH,
            )
            cp.start()
            cp.wait()

        mine = p_ref[pl.ds(d * m, m), :].astype(jnp.float32)
        out_ref[:, :] = mine + comm[N_DEV - 2].astype(jnp.float32)

        _exit_barrier(left, right)

    return pl.pallas_call(
        body,
        out_shape=jax.ShapeDtypeStruct((m, dmodel), jnp.float32),
        in_specs=[pl.BlockSpec(memory_space=pltpu.VMEM)],
        out_specs=pl.BlockSpec(memory_space=pltpu.VMEM),
        scratch_shapes=[
            pltpu.VMEM((N_DEV - 1, m, dmodel), jnp.bfloat16),
            pltpu.VMEM((m, dmodel), jnp.bfloat16),
            pltpu.SemaphoreType.DMA((N_DEV - 1,)),
            pltpu.SemaphoreType.DMA((N_DEV - 1,)),
        ],
        compiler_params=pltpu.CompilerParams(
            collective_id=1, vmem_limit_bytes=60 * 1024 * 1024,
        ),
    )(partial)


def kernel(x, router_W, route_idx, expert_W, shared_W):
    xb = x.astype(jnp.bfloat16)
    ew = expert_W.astype(jnp.bfloat16)
    sw = shared_W.astype(jnp.bfloat16)

    scores = x @ router_W
    scores = scores - scores.max(axis=-1, keepdims=True)
    probs = jnp.exp(scores)
    probs = probs / probs.sum(axis=-1, keepdims=True)
    ids = lax.broadcasted_iota(jnp.int32, probs.shape, 1)
    p_sel = jnp.sum(jnp.where(ids == route_idx, probs, 0.0), axis=1)
    meta = jnp.stack([route_idx[:, 0].astype(jnp.float32), p_sel])

    partial = _agc_kernel(xb, meta, ew, sw)
    return _rs_kernel(partial)
